# baseline (speedup 1.0000x reference)
import functools

import jax
import jax.numpy as jnp
from jax.experimental import pallas as pl
from jax.experimental.pallas import tpu as pltpu

DESIRED = 5.0
UNDESIRED = 1.0


def _body(x_hbm, t_ref, o_ref, acc_ref, buf_ref, sem,
          *, K, TB, NBUF, scale):
    def start(j):
        slot = jax.lax.rem(j, NBUF)
        pltpu.make_async_copy(
            x_hbm.at[pl.ds(j * TB, TB)],
            buf_ref.at[slot],
            sem.at[slot],
        ).start()

    def wait(j):
        slot = jax.lax.rem(j, NBUF)
        pltpu.make_async_copy(
            x_hbm.at[pl.ds(j * TB, TB)],
            buf_ref.at[slot],
            sem.at[slot],
        ).wait()

    for j in range(min(NBUF, K)):
        start(j)

    def step(j, carry):
        wait(j)

        slot = jax.lax.rem(j, NBUF)
        s = jnp.sum(buf_ref[slot], axis=0)

        @pl.when(j == 0)
        def _():
            acc_ref[...] = s

        @pl.when(j > 0)
        def _():
            acc_ref[...] += s

        @pl.when(j + NBUF < K)
        def _():
            start(j + NBUF)

        return carry

    jax.lax.fori_loop(0, K, step, 0, unroll=False)

    t = t_ref[...]
    oc = acc_ref[...]
    oc = jnp.where((t == DESIRED) & (oc > DESIRED), DESIRED, oc)
    oc = jnp.where((t == UNDESIRED) & (oc < UNDESIRED), UNDESIRED, oc)
    d = oc - t
    o_ref[0, 0] = jnp.sum(d * d) * scale


def kernel(output, target):
    T, N, C = output.shape
    TB = 2
    NBUF = 16
    assert T % TB == 0
    K = T // TB
    scale = 0.5 / T

    xt = jnp.transpose(output, (0, 2, 1))
    tt = target.T

    out = pl.pallas_call(
        functools.partial(_body, K=K, TB=TB, NBUF=NBUF, scale=scale),
        in_specs=[
            pl.BlockSpec(memory_space=pl.ANY),
            pl.BlockSpec(memory_space=pltpu.VMEM),
        ],
        out_specs=pl.BlockSpec(memory_space=pltpu.SMEM),
        out_shape=jax.ShapeDtypeStruct((1, 1), jnp.float32),
        scratch_shapes=[
            pltpu.VMEM((C, N), jnp.float32),
            pltpu.VMEM((NBUF, TB, C, N), jnp.float32),
            pltpu.SemaphoreType.DMA((NBUF,)),
        ],
    )(xt, tt)
    return out[0, 0]


# speedup vs baseline: 1.0206x; 1.0206x over previous
import functools

import jax
import jax.numpy as jnp
from jax.experimental import pallas as pl
from jax.experimental.pallas import tpu as pltpu

DESIRED = 5.0
UNDESIRED = 1.0


def _body(x_hbm, t_ref, o_ref, acc_ref, buf_ref, sem,
          *, K, TB, NBUF, scale):
    def start(j):
        slot = jax.lax.rem(j, NBUF)
        pltpu.make_async_copy(
            x_hbm.at[pl.ds(j * TB, TB)],
            buf_ref.at[slot],
            sem.at[slot],
        ).start()

    def wait(j):
        slot = jax.lax.rem(j, NBUF)
        pltpu.make_async_copy(
            x_hbm.at[pl.ds(j * TB, TB)],
            buf_ref.at[slot],
            sem.at[slot],
        ).wait()

    for j in range(min(NBUF, K)):
        start(j)

    def step(j, carry):
        wait(j)

        slot = jax.lax.rem(j, NBUF)
        s = jnp.sum(buf_ref[slot], axis=0)

        @pl.when(j == 0)
        def _():
            acc_ref[...] = s

        @pl.when(j > 0)
        def _():
            acc_ref[...] += s

        @pl.when(j + NBUF < K)
        def _():
            start(j + NBUF)

        return carry

    jax.lax.fori_loop(0, K, step, 0, unroll=False)

    t = t_ref[...]
    oc = acc_ref[...]
    oc = jnp.where((t == DESIRED) & (oc > DESIRED), DESIRED, oc)
    oc = jnp.where((t == UNDESIRED) & (oc < UNDESIRED), UNDESIRED, oc)
    d = oc - t
    o_ref[0, 0] = jnp.sum(d * d) * scale


def kernel(output, target):
    T, N, C = output.shape
    TB = 4
    NBUF = 6
    assert T % TB == 0
    K = T // TB
    scale = 0.5 / T

    xt = jnp.transpose(output, (0, 2, 1))
    tt = target.T

    out = pl.pallas_call(
        functools.partial(_body, K=K, TB=TB, NBUF=NBUF, scale=scale),
        in_specs=[
            pl.BlockSpec(memory_space=pl.ANY),
            pl.BlockSpec(memory_space=pltpu.VMEM),
        ],
        out_specs=pl.BlockSpec(memory_space=pltpu.SMEM),
        out_shape=jax.ShapeDtypeStruct((1, 1), jnp.float32),
        scratch_shapes=[
            pltpu.VMEM((C, N), jnp.float32),
            pltpu.VMEM((NBUF, TB, C, N), jnp.float32),
            pltpu.SemaphoreType.DMA((NBUF,)),
        ],
    )(xt, tt)
    return out[0, 0]


# speedup vs baseline: 1.0678x; 1.0463x over previous
import functools

import jax
import jax.numpy as jnp
from jax.experimental import pallas as pl
from jax.experimental.pallas import tpu as pltpu

DESIRED = 5.0
UNDESIRED = 1.0


def _body(x_hbm, t_ref, o_ref, acc_ref, buf_ref, sem,
          *, K, TB, NBUF, scale):
    def start(j):
        slot = jax.lax.rem(j, NBUF)
        pltpu.make_async_copy(
            x_hbm.at[pl.ds(j * TB, TB)],
            buf_ref.at[slot],
            sem.at[slot],
        ).start()

    def wait(j):
        slot = jax.lax.rem(j, NBUF)
        pltpu.make_async_copy(
            x_hbm.at[pl.ds(j * TB, TB)],
            buf_ref.at[slot],
            sem.at[slot],
        ).wait()

    for j in range(min(NBUF, K)):
        start(j)

    def step(j, carry):
        wait(j)

        slot = jax.lax.rem(j, NBUF)
        s = jnp.sum(buf_ref[slot], axis=0)

        @pl.when(j == 0)
        def _():
            acc_ref[...] = s

        @pl.when(j > 0)
        def _():
            acc_ref[...] += s

        @pl.when(j + NBUF < K)
        def _():
            start(j + NBUF)

        return carry

    jax.lax.fori_loop(0, K, step, 0, unroll=False)

    t = t_ref[...]
    oc = acc_ref[...]
    oc = jnp.where((t == DESIRED) & (oc > DESIRED), DESIRED, oc)
    oc = jnp.where((t == UNDESIRED) & (oc < UNDESIRED), UNDESIRED, oc)
    d = oc - t
    o_ref[0, 0] = jnp.sum(d * d) * scale


def kernel(output, target):
    T, N, C = output.shape
    TB = 4
    NBUF = 8
    assert T % TB == 0
    K = T // TB
    scale = 0.5 / T

    xt = jnp.transpose(output, (0, 2, 1))
    tt = target.T

    out = pl.pallas_call(
        functools.partial(_body, K=K, TB=TB, NBUF=NBUF, scale=scale),
        in_specs=[
            pl.BlockSpec(memory_space=pl.ANY),
            pl.BlockSpec(memory_space=pltpu.VMEM),
        ],
        out_specs=pl.BlockSpec(memory_space=pltpu.SMEM),
        out_shape=jax.ShapeDtypeStruct((1, 1), jnp.float32),
        scratch_shapes=[
            pltpu.VMEM((C, N), jnp.float32),
            pltpu.VMEM((NBUF, TB, C, N), jnp.float32),
            pltpu.SemaphoreType.DMA((NBUF,)),
        ],
    )(xt, tt)
    return out[0, 0]
